# Initial kernel scaffold; baseline (speedup 1.0000x reference)
import jax
import jax.numpy as jnp
from jax.experimental import pallas as pl


def kernel(x, edge_weight, edge_attr, pre_W, pre_b, mlp_W1, mlp_b1, mlp_W2, mlp_b2, lin1_W, lin2_W, lin2_b, blk_W, blk_b, bn_g, bn_b, post_W, post_b, out_W, out_b, edge_index, batch):
    raise NotImplementedError("write your pallas kernel here")


# speedup vs baseline: 2.8027x; 2.8027x over previous
import functools

import jax
import jax.numpy as jnp
from jax import lax
from jax.experimental import pallas as pl
from jax.experimental.pallas import tpu as pltpu
from jax.experimental.pallas import tpu_sc as plsc

_N = 10000
_E = 320000
_D = 128
_H = 64
_G = 16
_B = 32
_L = 3
_CUTOFF = 8.0
_LOG2 = 0.6931471805599453

_NW = 32
_EPW = _E // _NW
_CHUNK = 80
_SUB = 5
_SUP_E = _CHUNK * _SUB
_NSUP = _EPW // _SUP_E
_NCH = _EPW // _CHUNK
_NP = 10240
_RPT = _NP // 16
_CPR = 128


def _ssp(v):
    return jnp.maximum(v, 0.0) + jnp.log1p(jnp.exp(-jnp.abs(v))) - _LOG2


_BE = 2000


def _edge_body(ew_ref, ea_ref, w1_ref, b1_ref, w2_ref, b2_ref,
               wf0_ref, wf1_ref, wf2_ref):
    ew = ew_ref[...]
    c = 0.5 * (jnp.cos(ew * (jnp.pi / _CUTOFF)) + 1.0)
    ea = ea_ref[...]
    t = jnp.dot(ea, w1_ref[...], preferred_element_type=jnp.float32) + b1_ref[...]
    s = _ssp(t)
    wf = jnp.dot(s, w2_ref[...], preferred_element_type=jnp.float32) + b2_ref[...]
    wf = wf * c
    wf0_ref[...] = wf[:, 0:_H]
    wf1_ref[...] = wf[:, _H:2 * _H]
    wf2_ref[...] = wf[:, 2 * _H:3 * _H]


_edge_call = pl.pallas_call(
    _edge_body,
    grid=(_E // _BE,),
    in_specs=[
        pl.BlockSpec((_BE, 1), lambda i: (i, 0)),
        pl.BlockSpec((_BE, _G), lambda i: (i, 0)),
        pl.BlockSpec((_G, _L * _H), lambda i: (0, 0)),
        pl.BlockSpec((1, _L * _H), lambda i: (0, 0)),
        pl.BlockSpec((_L * _H, _L * _H), lambda i: (0, 0)),
        pl.BlockSpec((1, _L * _H), lambda i: (0, 0)),
    ],
    out_specs=[pl.BlockSpec((_BE, _H), lambda i: (i, 0))] * 3,
    out_shape=[jax.ShapeDtypeStruct((_E, _H), jnp.float32)] * 3,
)


def _pre_body(x_ref, pw_ref, pb_ref, l1_ref, h_ref, hs_ref):
    h = jax.nn.relu(jnp.dot(x_ref[...], pw_ref[...],
                            preferred_element_type=jnp.float32) + pb_ref[...])
    h_ref[...] = h
    hs_ref[...] = jnp.dot(h, l1_ref[...], preferred_element_type=jnp.float32)


_pre_call = pl.pallas_call(
    _pre_body,
    out_shape=[jax.ShapeDtypeStruct((_N, _H), jnp.float32)] * 2,
)


def _node_update(h, part, l2w, l2b, bw, bb, g, b):
    agg = (part[0] + part[1])[:_N]
    c = _ssp(jnp.dot(agg, l2w, preferred_element_type=jnp.float32) + l2b)
    c = jnp.dot(c, bw, preferred_element_type=jnp.float32) + bb
    hn = h + c
    mu = jnp.mean(hn, axis=0, keepdims=True)
    var = jnp.mean((hn - mu) ** 2, axis=0, keepdims=True)
    return (hn - mu) * jax.lax.rsqrt(var + 1e-5) * g + b


def _node_body(h_ref, part_ref, l2w_ref, l2b_ref, bw_ref, bb_ref,
               g_ref, b_ref, l1n_ref, h_out, hs_out):
    hn = _node_update(h_ref[...], part_ref[...], l2w_ref[...], l2b_ref[...],
                     bw_ref[...], bb_ref[...], g_ref[...], b_ref[...])
    h_out[...] = hn
    hs_out[...] = jnp.dot(hn, l1n_ref[...], preferred_element_type=jnp.float32)


_node_call = pl.pallas_call(
    _node_body,
    out_shape=[jax.ShapeDtypeStruct((_N, _H), jnp.float32)] * 2,
)


def _final_body(h_ref, part_ref, l2w_ref, l2b_ref, bw_ref, bb_ref,
                g_ref, b_ref, batch_ref, pw_ref, pb_ref, ow_ref, ob_ref,
                o_ref):
    hn = _node_update(h_ref[...], part_ref[...], l2w_ref[...], l2b_ref[...],
                     bw_ref[...], bb_ref[...], g_ref[...], b_ref[...])
    seg = lax.broadcasted_iota(jnp.int32, (_B, _N), 0)
    oht = (seg == batch_ref[...]).astype(jnp.float32)
    counts = jnp.sum(oht, axis=1, keepdims=True)
    pooled = jnp.dot(oht, hn, preferred_element_type=jnp.float32)
    pooled = pooled / jnp.maximum(counts, 1.0)
    o = jax.nn.relu(jnp.dot(pooled, pw_ref[...],
                            preferred_element_type=jnp.float32) + pb_ref[...])
    o_ref[...] = jnp.dot(o, ow_ref[...], preferred_element_type=jnp.float32) + ob_ref[...]


_final_call = pl.pallas_call(
    _final_body,
    out_shape=jax.ShapeDtypeStruct((_B, 1), jnp.float32),
)


def _sc_body(hs, src3, dst3, wf, out, idx_s, idx_d, rows, wfv, tmp,
             agg_sh, sem_g, sem_w):
    c = lax.axis_index("c")
    s = lax.axis_index("s")
    w = c * 16 + s

    def zbody(j, _):
        r = j // 4
        q = j % 4
        tmp[r, pl.ds(q * 16, 16)] = jnp.zeros((16,), jnp.float32)
        return 0
    lax.fori_loop(0, _CPR * 4, zbody, 0)
    row0 = s * _RPT
    for t in range(_RPT // _CPR):
        pltpu.sync_copy(tmp, agg_sh.at[pl.ds(row0 + t * _CPR, _CPR)])
    plsc.subcore_barrier()

    pltpu.sync_copy(src3.at[w], idx_s)
    pltpu.sync_copy(dst3.at[w], idx_d)

    ebase = w * _EPW

    def super_body(gi, _):
        sbase = ebase + gi * _SUP_E
        cp_w = pltpu.async_copy(wf.at[pl.ds(sbase, _SUP_E)], wfv, sem_w)
        cps = [
            pltpu.async_copy(hs.at[idx_s.at[gi * _SUB + t]],
                             rows.at[pl.ds(t * _CHUNK, _CHUNK)], sem_g)
            for t in range(_SUB)
        ]
        for cp in cps:
            cp.wait()
        cp_w.wait()

        def mbody(j, _):
            for q in range(4):
                sl = pl.ds(q * 16, 16)
                rows[j, sl] = rows[j, sl] * wfv[j, sl]
            return 0
        lax.fori_loop(0, _SUP_E, mbody, 0)

        for t in range(_SUB):
            pltpu.sync_copy(rows.at[pl.ds(t * _CHUNK, _CHUNK)],
                            agg_sh.at[idx_d.at[gi * _SUB + t]], add=True)
        return 0

    lax.fori_loop(0, _NSUP, super_body, 0)
    plsc.subcore_barrier()

    for t in range(_RPT // _CPR):
        pltpu.sync_copy(agg_sh.at[pl.ds(row0 + t * _CPR, _CPR)], tmp)
        pltpu.sync_copy(tmp, out.at[c, pl.ds(row0 + t * _CPR, _CPR)])


@functools.cache
def _sc_call():
  return pl.kernel(
    _sc_body,
    out_type=jax.ShapeDtypeStruct((2, _NP, _H), jnp.float32),
    mesh=plsc.VectorSubcoreMesh(core_axis_name="c", subcore_axis_name="s",
                                num_cores=2, num_subcores=16),
    compiler_params=pltpu.CompilerParams(use_tc_tiling_on_sc=False),
    scratch_types=[
        pltpu.VMEM((_NCH, _CHUNK), jnp.int32),
        pltpu.VMEM((_NCH, _CHUNK), jnp.int32),
        pltpu.VMEM((_SUP_E, _H), jnp.float32),
        pltpu.VMEM((_SUP_E, _H), jnp.float32),
        pltpu.VMEM((_CPR, _H), jnp.float32),
        pltpu.VMEM_SHARED((_NP, _H), jnp.float32),
        pltpu.SemaphoreType.DMA,
        pltpu.SemaphoreType.DMA,
    ],
  )


def kernel(x, edge_weight, edge_attr, pre_W, pre_b, mlp_W1, mlp_b1, mlp_W2,
           mlp_b2, lin1_W, lin2_W, lin2_b, blk_W, blk_b, bn_g, bn_b, post_W,
           post_b, out_W, out_b, edge_index, batch):
    src3 = edge_index[0].reshape(_NW, _NCH, _CHUNK)
    dst3 = edge_index[1].reshape(_NW, _NCH, _CHUNK)

    w1c = jnp.concatenate([mlp_W1[0], mlp_W1[1], mlp_W1[2]], axis=1)
    b1c = mlp_b1.reshape(1, _L * _H)
    z = jnp.zeros((_H, _H), jnp.float32)
    w2bd = jnp.block([[mlp_W2[0], z, z], [z, mlp_W2[1], z], [z, z, mlp_W2[2]]])
    b2c = mlp_b2.reshape(1, _L * _H)

    wfs = _edge_call(edge_weight.reshape(_E, 1), edge_attr, w1c, b1c, w2bd, b2c)

    h, hs = _pre_call(x, pre_W, pre_b.reshape(1, _H), lin1_W[0])

    for i in range(_L):
        part = _sc_call()(hs, src3, dst3, wfs[i])
        args = (h, part, lin2_W[i], lin2_b[i].reshape(1, _H), blk_W[i],
                blk_b[i].reshape(1, _H), bn_g[i].reshape(1, _H),
                bn_b[i].reshape(1, _H))
        if i < _L - 1:
            h, hs = _node_call(*args, lin1_W[i + 1])
        else:
            o = _final_call(*args, batch.reshape(1, _N), post_W,
                            post_b.reshape(1, _H), out_W, out_b.reshape(1, 1))
    return o.reshape(-1)


# speedup vs baseline: 3.9959x; 1.4257x over previous
import functools

import jax
import jax.numpy as jnp
from jax import lax
from jax.experimental import pallas as pl
from jax.experimental.pallas import tpu as pltpu
from jax.experimental.pallas import tpu_sc as plsc

_N = 10000
_E = 320000
_D = 128
_H = 64
_G = 16
_B = 32
_L = 3
_CUTOFF = 8.0
_LOG2 = 0.6931471805599453

_NW = 32
_EPW = _E // _NW
_CHUNK = 80
_SUB = 5
_SUP_E = _CHUNK * _SUB
_NSUP = _EPW // _SUP_E
_NCH = _EPW // _CHUNK
_NP = 10240
_RPT = _NP // 16
_CPR = 128


def _ssp(v):
    return jnp.log(jnp.exp(v) + 1.0) - _LOG2


_BE = 2000


def _cenv_body(ew_ref, c_ref):
    c_ref[...] = 0.5 * (jnp.cos(ew_ref[...] * (jnp.pi / _CUTOFF)) + 1.0)


_cenv_call = pl.pallas_call(
    _cenv_body,
    out_shape=jax.ShapeDtypeStruct((_E // 128, 128), jnp.float32),
)


def _edge_body(c_ref, ea_ref, w1_ref, b1_ref, w2_ref, b2_ref,
               wf0_ref, wf1_ref, wf2_ref):
    c = c_ref[...]
    ea = ea_ref[...]
    t = jnp.dot(ea, w1_ref[...], preferred_element_type=jnp.float32) + b1_ref[...]
    s = _ssp(t)
    wf = jnp.dot(s, w2_ref[...], preferred_element_type=jnp.float32) + b2_ref[...]
    wf = wf * c
    wf0_ref[...] = wf[:, 0:_H]
    wf1_ref[...] = wf[:, _H:2 * _H]
    wf2_ref[...] = wf[:, 2 * _H:3 * _H]


_edge_call = pl.pallas_call(
    _edge_body,
    grid=(_E // _BE,),
    in_specs=[
        pl.BlockSpec((_BE, 1), lambda i: (i, 0)),
        pl.BlockSpec((_BE, _G), lambda i: (i, 0)),
        pl.BlockSpec((_G, _L * _H), lambda i: (0, 0)),
        pl.BlockSpec((1, _L * _H), lambda i: (0, 0)),
        pl.BlockSpec((_L * _H, _L * _H), lambda i: (0, 0)),
        pl.BlockSpec((1, _L * _H), lambda i: (0, 0)),
    ],
    out_specs=[pl.BlockSpec((_BE, _H), lambda i: (i, 0))] * 3,
    out_shape=[jax.ShapeDtypeStruct((_E, _H), jnp.float32)] * 3,
)


def _pre_body(x_ref, pw_ref, pb_ref, l1_ref, h_ref, hs_ref):
    h = jax.nn.relu(jnp.dot(x_ref[...], pw_ref[...],
                            preferred_element_type=jnp.float32) + pb_ref[...])
    h_ref[...] = h
    hs_ref[...] = jnp.dot(h, l1_ref[...], preferred_element_type=jnp.float32)


_pre_call = pl.pallas_call(
    _pre_body,
    out_shape=[jax.ShapeDtypeStruct((_N, _H), jnp.float32)] * 2,
)


def _node_update(h, part, l2w, l2b, bw, bb, g, b):
    agg = (part[0] + part[1])[:_N]
    c = _ssp(jnp.dot(agg, l2w, preferred_element_type=jnp.float32) + l2b)
    c = jnp.dot(c, bw, preferred_element_type=jnp.float32) + bb
    hn = h + c
    mu = jnp.mean(hn, axis=0, keepdims=True)
    var = jnp.mean((hn - mu) ** 2, axis=0, keepdims=True)
    return (hn - mu) * jax.lax.rsqrt(var + 1e-5) * g + b


def _node_body(h_ref, part_ref, l2w_ref, l2b_ref, bw_ref, bb_ref,
               g_ref, b_ref, l1n_ref, h_out, hs_out):
    hn = _node_update(h_ref[...], part_ref[...], l2w_ref[...], l2b_ref[...],
                     bw_ref[...], bb_ref[...], g_ref[...], b_ref[...])
    h_out[...] = hn
    hs_out[...] = jnp.dot(hn, l1n_ref[...], preferred_element_type=jnp.float32)


_node_call = pl.pallas_call(
    _node_body,
    out_shape=[jax.ShapeDtypeStruct((_N, _H), jnp.float32)] * 2,
)


def _final_body(h_ref, part_ref, l2w_ref, l2b_ref, bw_ref, bb_ref,
                g_ref, b_ref, batch_ref, pw_ref, pb_ref, ow_ref, ob_ref,
                o_ref):
    hn = _node_update(h_ref[...], part_ref[...], l2w_ref[...], l2b_ref[...],
                     bw_ref[...], bb_ref[...], g_ref[...], b_ref[...])
    seg = lax.broadcasted_iota(jnp.int32, (_B, _N), 0)
    oht = (seg == batch_ref[...]).astype(jnp.float32)
    counts = jnp.sum(oht, axis=1, keepdims=True)
    pooled = jnp.dot(oht, hn, preferred_element_type=jnp.float32)
    pooled = pooled / jnp.maximum(counts, 1.0)
    o = jax.nn.relu(jnp.dot(pooled, pw_ref[...],
                            preferred_element_type=jnp.float32) + pb_ref[...])
    o_ref[...] = jnp.dot(o, ow_ref[...], preferred_element_type=jnp.float32) + ob_ref[...]


_final_call = pl.pallas_call(
    _final_body,
    out_shape=jax.ShapeDtypeStruct((_B, 1), jnp.float32),
)


def _sc_body(hs, src3, dst3, wf, out, idx_s, idx_d, rows, wfv, tmp,
             agg_sh, sem_g, sem_w):
    c = lax.axis_index("c")
    s = lax.axis_index("s")
    w = c * 16 + s

    def zbody(j, _):
        r = j // 4
        q = j % 4
        tmp[r, pl.ds(q * 16, 16)] = jnp.zeros((16,), jnp.float32)
        return 0
    lax.fori_loop(0, _CPR * 4, zbody, 0)
    row0 = s * _RPT
    for t in range(_RPT // _CPR):
        pltpu.sync_copy(tmp, agg_sh.at[pl.ds(row0 + t * _CPR, _CPR)])
    plsc.subcore_barrier()

    pltpu.sync_copy(src3.at[w], idx_s)
    pltpu.sync_copy(dst3.at[w], idx_d)

    ebase = w * _EPW

    def super_body(gi, _):
        sbase = ebase + gi * _SUP_E
        cp_w = pltpu.async_copy(wf.at[pl.ds(sbase, _SUP_E)], wfv, sem_w)
        cps = [
            pltpu.async_copy(hs.at[idx_s.at[gi * _SUB + t]],
                             rows.at[pl.ds(t * _CHUNK, _CHUNK)], sem_g)
            for t in range(_SUB)
        ]
        for cp in cps:
            cp.wait()
        cp_w.wait()

        def mbody(j, _):
            for q in range(4):
                sl = pl.ds(q * 16, 16)
                rows[j, sl] = rows[j, sl] * wfv[j, sl]
            return 0
        lax.fori_loop(0, _SUP_E, mbody, 0)

        for t in range(_SUB):
            pltpu.sync_copy(rows.at[pl.ds(t * _CHUNK, _CHUNK)],
                            agg_sh.at[idx_d.at[gi * _SUB + t]], add=True)
        return 0

    lax.fori_loop(0, _NSUP, super_body, 0)
    plsc.subcore_barrier()

    for t in range(_RPT // _CPR):
        pltpu.sync_copy(agg_sh.at[pl.ds(row0 + t * _CPR, _CPR)], tmp)
        pltpu.sync_copy(tmp, out.at[c, pl.ds(row0 + t * _CPR, _CPR)])


@functools.cache
def _sc_call():
  return pl.kernel(
    _sc_body,
    out_type=jax.ShapeDtypeStruct((2, _NP, _H), jnp.float32),
    mesh=plsc.VectorSubcoreMesh(core_axis_name="c", subcore_axis_name="s",
                                num_cores=2, num_subcores=16),
    compiler_params=pltpu.CompilerParams(use_tc_tiling_on_sc=False),
    scratch_types=[
        pltpu.VMEM((_NCH, _CHUNK), jnp.int32),
        pltpu.VMEM((_NCH, _CHUNK), jnp.int32),
        pltpu.VMEM((_SUP_E, _H), jnp.float32),
        pltpu.VMEM((_SUP_E, _H), jnp.float32),
        pltpu.VMEM((_CPR, _H), jnp.float32),
        pltpu.VMEM_SHARED((_NP, _H), jnp.float32),
        pltpu.SemaphoreType.DMA,
        pltpu.SemaphoreType.DMA,
    ],
  )


def kernel(x, edge_weight, edge_attr, pre_W, pre_b, mlp_W1, mlp_b1, mlp_W2,
           mlp_b2, lin1_W, lin2_W, lin2_b, blk_W, blk_b, bn_g, bn_b, post_W,
           post_b, out_W, out_b, edge_index, batch):
    src3 = edge_index[0].reshape(_NW, _NCH, _CHUNK)
    dst3 = edge_index[1].reshape(_NW, _NCH, _CHUNK)

    w1c = jnp.concatenate([mlp_W1[0], mlp_W1[1], mlp_W1[2]], axis=1)
    b1c = mlp_b1.reshape(1, _L * _H)
    z = jnp.zeros((_H, _H), jnp.float32)
    w2bd = jnp.block([[mlp_W2[0], z, z], [z, mlp_W2[1], z], [z, z, mlp_W2[2]]])
    b2c = mlp_b2.reshape(1, _L * _H)

    cenv = _cenv_call(edge_weight.reshape(_E // 128, 128)).reshape(_E, 1)
    wfs = _edge_call(cenv, edge_attr, w1c, b1c, w2bd, b2c)

    h, hs = _pre_call(x, pre_W, pre_b.reshape(1, _H), lin1_W[0])

    for i in range(_L):
        part = _sc_call()(hs, src3, dst3, wfs[i])
        args = (h, part, lin2_W[i], lin2_b[i].reshape(1, _H), blk_W[i],
                blk_b[i].reshape(1, _H), bn_g[i].reshape(1, _H),
                bn_b[i].reshape(1, _H))
        if i < _L - 1:
            h, hs = _node_call(*args, lin1_W[i + 1])
        else:
            o = _final_call(*args, batch.reshape(1, _N), post_W,
                            post_b.reshape(1, _H), out_W, out_b.reshape(1, 1))
    return o.reshape(-1)


# speedup vs baseline: 5.8671x; 1.4683x over previous
import functools

import jax
import jax.numpy as jnp
from jax import lax
from jax.experimental import pallas as pl
from jax.experimental.pallas import tpu as pltpu
from jax.experimental.pallas import tpu_sc as plsc

_N = 10000
_E = 320000
_D = 128
_H = 64
_G = 16
_B = 32
_L = 3
_CUTOFF = 8.0
_LOG2 = 0.6931471805599453

_NW = 32
_EPW = _E // _NW
_CHUNK = 80
_SUB = 5
_SUP_E = _CHUNK * _SUB
_NSUP = _EPW // _SUP_E
_NCH = _EPW // _CHUNK
_NP = 10240
_RPT = _NP // 16
_CPR = 128


def _ssp(v):
    return jnp.log(jnp.exp(v) + 1.0) - _LOG2


_BE = 2560


def _edge_body(ew_ref, ea_ref, w1_ref, b1_ref, w2_ref, b2_ref,
               wfa_ref, wfb_ref):
    c = 0.5 * (jnp.cos(ew_ref[...] * (jnp.pi / _CUTOFF)) + 1.0)
    c = c.reshape(_BE, 1)
    ea = jnp.transpose(ea_ref[...])
    t = jnp.dot(ea, w1_ref[...], preferred_element_type=jnp.float32) + b1_ref[...]
    s = _ssp(t)
    wf = jnp.dot(s, w2_ref[...], preferred_element_type=jnp.float32) + b2_ref[...]
    wf = wf * c
    wfa_ref[...] = wf[:, 0:2 * _H]
    wfb_ref[...] = wf[:, _H:3 * _H]


_edge_call = pl.pallas_call(
    _edge_body,
    grid=(_E // _BE,),
    in_specs=[
        pl.BlockSpec((1, _BE), lambda i: (0, i)),
        pl.BlockSpec((_G, _BE), lambda i: (0, i)),
        pl.BlockSpec((_G, _L * _H), lambda i: (0, 0)),
        pl.BlockSpec((1, _L * _H), lambda i: (0, 0)),
        pl.BlockSpec((_L * _H, _L * _H), lambda i: (0, 0)),
        pl.BlockSpec((1, _L * _H), lambda i: (0, 0)),
    ],
    out_specs=[pl.BlockSpec((_BE, 2 * _H), lambda i: (i, 0))] * 2,
    out_shape=[jax.ShapeDtypeStruct((_E, 2 * _H), jnp.float32)] * 2,
)


def _pre_body(x_ref, pw_ref, pb_ref, l1_ref, h_ref, hs_ref):
    h = jax.nn.relu(jnp.dot(x_ref[...], pw_ref[...],
                            preferred_element_type=jnp.float32) + pb_ref[...])
    h_ref[...] = h
    hs_ref[...] = jnp.dot(h, l1_ref[...], preferred_element_type=jnp.float32)


_pre_call = pl.pallas_call(
    _pre_body,
    out_shape=[jax.ShapeDtypeStruct((_N, _H), jnp.float32)] * 2,
)


def _node_update(h, part, l2w, l2b, bw, bb, g, b):
    agg = (part[0] + part[1])[:_N]
    c = _ssp(jnp.dot(agg, l2w, preferred_element_type=jnp.float32) + l2b)
    c = jnp.dot(c, bw, preferred_element_type=jnp.float32) + bb
    hn = h + c
    mu = jnp.mean(hn, axis=0, keepdims=True)
    var = jnp.mean((hn - mu) ** 2, axis=0, keepdims=True)
    return (hn - mu) * jax.lax.rsqrt(var + 1e-5) * g + b


def _node_body(h_ref, part_ref, l2w_ref, l2b_ref, bw_ref, bb_ref,
               g_ref, b_ref, l1n_ref, h_out, hs_out):
    hn = _node_update(h_ref[...], part_ref[...], l2w_ref[...], l2b_ref[...],
                     bw_ref[...], bb_ref[...], g_ref[...], b_ref[...])
    h_out[...] = hn
    hs_out[...] = jnp.dot(hn, l1n_ref[...], preferred_element_type=jnp.float32)


_node_call = pl.pallas_call(
    _node_body,
    out_shape=[jax.ShapeDtypeStruct((_N, _H), jnp.float32)] * 2,
)


def _final_body(h_ref, part_ref, l2w_ref, l2b_ref, bw_ref, bb_ref,
                g_ref, b_ref, batch_ref, pw_ref, pb_ref, ow_ref, ob_ref,
                o_ref):
    hn = _node_update(h_ref[...], part_ref[...], l2w_ref[...], l2b_ref[...],
                     bw_ref[...], bb_ref[...], g_ref[...], b_ref[...])
    seg = lax.broadcasted_iota(jnp.int32, (_B, _N), 0)
    oht = (seg == batch_ref[...]).astype(jnp.float32)
    counts = jnp.sum(oht, axis=1, keepdims=True)
    pooled = jnp.dot(oht, hn, preferred_element_type=jnp.float32)
    pooled = pooled / jnp.maximum(counts, 1.0)
    o = jax.nn.relu(jnp.dot(pooled, pw_ref[...],
                            preferred_element_type=jnp.float32) + pb_ref[...])
    o_ref[...] = jnp.dot(o, ow_ref[...], preferred_element_type=jnp.float32) + ob_ref[...]


_final_call = pl.pallas_call(
    _final_body,
    out_shape=jax.ShapeDtypeStruct((_B, 1), jnp.float32),
)


def _make_sc_body(off):
  def _sc_body(hs, src3, dst3, wf, out, idx_s, idx_d, rows, wfv, tmp,
               agg_sh, sem_g, sem_w):
    c = lax.axis_index("c")
    s = lax.axis_index("s")
    w = c * 16 + s

    def zbody(j, _):
        r = j // 4
        q = j % 4
        tmp[r, pl.ds(q * 16, 16)] = jnp.zeros((16,), jnp.float32)
        return 0
    lax.fori_loop(0, _CPR * 4, zbody, 0)
    row0 = s * _RPT
    for t in range(_RPT // _CPR):
        pltpu.sync_copy(tmp, agg_sh.at[pl.ds(row0 + t * _CPR, _CPR)])
    plsc.subcore_barrier()

    pltpu.sync_copy(src3.at[w], idx_s)
    pltpu.sync_copy(dst3.at[w], idx_d)

    ebase = w * _EPW

    def super_body(gi, _):
        sbase = ebase + gi * _SUP_E
        cp_w = pltpu.async_copy(wf.at[pl.ds(sbase, _SUP_E), pl.ds(off, _H)],
                                wfv, sem_w)
        cps = [
            pltpu.async_copy(hs.at[idx_s.at[gi * _SUB + t]],
                             rows.at[pl.ds(t * _CHUNK, _CHUNK)], sem_g)
            for t in range(_SUB)
        ]
        for cp in cps:
            cp.wait()
        cp_w.wait()

        def mbody(j, _):
            for q in range(4):
                sl = pl.ds(q * 16, 16)
                rows[j, sl] = rows[j, sl] * wfv[j, sl]
            return 0
        lax.fori_loop(0, _SUP_E, mbody, 0)

        for t in range(_SUB):
            pltpu.sync_copy(rows.at[pl.ds(t * _CHUNK, _CHUNK)],
                            agg_sh.at[idx_d.at[gi * _SUB + t]], add=True)
        return 0

    lax.fori_loop(0, _NSUP, super_body, 0)
    plsc.subcore_barrier()

    for t in range(_RPT // _CPR):
        pltpu.sync_copy(agg_sh.at[pl.ds(row0 + t * _CPR, _CPR)], tmp)
        pltpu.sync_copy(tmp, out.at[c, pl.ds(row0 + t * _CPR, _CPR)])

  return _sc_body


@functools.cache
def _sc_call(off):
  return pl.kernel(
    _make_sc_body(off),
    out_type=jax.ShapeDtypeStruct((2, _NP, _H), jnp.float32),
    mesh=plsc.VectorSubcoreMesh(core_axis_name="c", subcore_axis_name="s",
                                num_cores=2, num_subcores=16),
    compiler_params=pltpu.CompilerParams(use_tc_tiling_on_sc=False),
    scratch_types=[
        pltpu.VMEM((_NCH, _CHUNK), jnp.int32),
        pltpu.VMEM((_NCH, _CHUNK), jnp.int32),
        pltpu.VMEM((_SUP_E, _H), jnp.float32),
        pltpu.VMEM((_SUP_E, _H), jnp.float32),
        pltpu.VMEM((_CPR, _H), jnp.float32),
        pltpu.VMEM_SHARED((_NP, _H), jnp.float32),
        pltpu.SemaphoreType.DMA,
        pltpu.SemaphoreType.DMA,
    ],
  )


def kernel(x, edge_weight, edge_attr, pre_W, pre_b, mlp_W1, mlp_b1, mlp_W2,
           mlp_b2, lin1_W, lin2_W, lin2_b, blk_W, blk_b, bn_g, bn_b, post_W,
           post_b, out_W, out_b, edge_index, batch):
    src3 = edge_index[0].reshape(_NW, _NCH, _CHUNK)
    dst3 = edge_index[1].reshape(_NW, _NCH, _CHUNK)

    w1c = jnp.concatenate([mlp_W1[0], mlp_W1[1], mlp_W1[2]], axis=1)
    b1c = mlp_b1.reshape(1, _L * _H)
    z = jnp.zeros((_H, _H), jnp.float32)
    w2bd = jnp.block([[mlp_W2[0], z, z], [z, mlp_W2[1], z], [z, z, mlp_W2[2]]])
    b2c = mlp_b2.reshape(1, _L * _H)

    wfa, wfb = _edge_call(edge_weight.reshape(1, _E), edge_attr.T, w1c, b1c,
                          w2bd, b2c)
    wf_src = ((wfa, 0), (wfa, _H), (wfb, _H))

    h, hs = _pre_call(x, pre_W, pre_b.reshape(1, _H), lin1_W[0])

    for i in range(_L):
        arr, off = wf_src[i]
        part = _sc_call(off)(hs, src3, dst3, arr)
        args = (h, part, lin2_W[i], lin2_b[i].reshape(1, _H), blk_W[i],
                blk_b[i].reshape(1, _H), bn_g[i].reshape(1, _H),
                bn_b[i].reshape(1, _H))
        if i < _L - 1:
            h, hs = _node_call(*args, lin1_W[i + 1])
        else:
            o = _final_call(*args, batch.reshape(1, _N), post_W,
                            post_b.reshape(1, _H), out_W, out_b.reshape(1, 1))
    return o.reshape(-1)


# speedup vs baseline: 6.8369x; 1.1653x over previous
import functools

import jax
import jax.numpy as jnp
from jax import lax
from jax.experimental import pallas as pl
from jax.experimental.pallas import tpu as pltpu
from jax.experimental.pallas import tpu_sc as plsc

_N = 10000
_E = 320000
_D = 128
_H = 64
_G = 16
_B = 32
_L = 3
_CUTOFF = 8.0
_LOG2 = 0.6931471805599453

_NW = 32
_EPW = _E // _NW
_CHUNK = 40
_SUB = 5
_SUP_E = _CHUNK * _SUB
_NSUP = _EPW // _SUP_E
_NCH = _EPW // _CHUNK
_NP = 10240
_RPT = _NP // 16
_CPR = 64


def _ssp(v):
    return jnp.log(jnp.exp(v) + 1.0) - _LOG2


_BE = 2560


def _edge_body(ew_ref, ea_ref, w1_ref, b1_ref, w2_ref, b2_ref,
               wfa_ref, wfb_ref):
    c = 0.5 * (jnp.cos(ew_ref[...] * (jnp.pi / _CUTOFF)) + 1.0)
    c = c.reshape(_BE, 1)
    ea = jnp.transpose(ea_ref[...])
    t = jnp.dot(ea, w1_ref[...], preferred_element_type=jnp.float32) + b1_ref[...]
    s = _ssp(t)
    wf = jnp.dot(s, w2_ref[...], preferred_element_type=jnp.float32) + b2_ref[...]
    wf = wf * c
    wfa_ref[...] = wf[:, 0:2 * _H]
    wfb_ref[...] = wf[:, _H:3 * _H]


_edge_call = pl.pallas_call(
    _edge_body,
    grid=(_E // _BE,),
    in_specs=[
        pl.BlockSpec((1, _BE), lambda i: (0, i)),
        pl.BlockSpec((_G, _BE), lambda i: (0, i)),
        pl.BlockSpec((_G, _L * _H), lambda i: (0, 0)),
        pl.BlockSpec((1, _L * _H), lambda i: (0, 0)),
        pl.BlockSpec((_L * _H, _L * _H), lambda i: (0, 0)),
        pl.BlockSpec((1, _L * _H), lambda i: (0, 0)),
    ],
    out_specs=[pl.BlockSpec((_BE, 2 * _H), lambda i: (i, 0))] * 2,
    out_shape=[jax.ShapeDtypeStruct((_E, 2 * _H), jnp.float32)] * 2,
)


def _pre_body(x_ref, pw_ref, pb_ref, l1_ref, h_ref, hs_ref):
    h = jax.nn.relu(jnp.dot(x_ref[...], pw_ref[...],
                            preferred_element_type=jnp.float32) + pb_ref[...])
    h_ref[...] = h
    hs_ref[...] = jnp.dot(h, l1_ref[...], preferred_element_type=jnp.float32)


_pre_call = pl.pallas_call(
    _pre_body,
    out_shape=[jax.ShapeDtypeStruct((_N, _H), jnp.float32)] * 2,
)


def _node_update(h, part, l2w, l2b, bw, bb, g, b):
    agg = (part[0] + part[1])[:_N]
    c = _ssp(jnp.dot(agg, l2w, preferred_element_type=jnp.float32) + l2b)
    c = jnp.dot(c, bw, preferred_element_type=jnp.float32) + bb
    hn = h + c
    mu = jnp.mean(hn, axis=0, keepdims=True)
    var = jnp.mean((hn - mu) ** 2, axis=0, keepdims=True)
    return (hn - mu) * jax.lax.rsqrt(var + 1e-5) * g + b


def _node_body(h_ref, part_ref, l2w_ref, l2b_ref, bw_ref, bb_ref,
               g_ref, b_ref, l1n_ref, h_out, hs_out):
    hn = _node_update(h_ref[...], part_ref[...], l2w_ref[...], l2b_ref[...],
                     bw_ref[...], bb_ref[...], g_ref[...], b_ref[...])
    h_out[...] = hn
    hs_out[...] = jnp.dot(hn, l1n_ref[...], preferred_element_type=jnp.float32)


_node_call = pl.pallas_call(
    _node_body,
    out_shape=[jax.ShapeDtypeStruct((_N, _H), jnp.float32)] * 2,
)


def _final_body(h_ref, part_ref, l2w_ref, l2b_ref, bw_ref, bb_ref,
                g_ref, b_ref, batch_ref, pw_ref, pb_ref, ow_ref, ob_ref,
                o_ref):
    hn = _node_update(h_ref[...], part_ref[...], l2w_ref[...], l2b_ref[...],
                     bw_ref[...], bb_ref[...], g_ref[...], b_ref[...])
    seg = lax.broadcasted_iota(jnp.int32, (_B, _N), 0)
    oht = (seg == batch_ref[...]).astype(jnp.float32)
    counts = jnp.sum(oht, axis=1, keepdims=True)
    pooled = jnp.dot(oht, hn, preferred_element_type=jnp.float32)
    pooled = pooled / jnp.maximum(counts, 1.0)
    o = jax.nn.relu(jnp.dot(pooled, pw_ref[...],
                            preferred_element_type=jnp.float32) + pb_ref[...])
    o_ref[...] = jnp.dot(o, ow_ref[...], preferred_element_type=jnp.float32) + ob_ref[...]


_final_call = pl.pallas_call(
    _final_body,
    out_shape=jax.ShapeDtypeStruct((_B, 1), jnp.float32),
)


def _make_sc_body(off):
  def _sc_body(hs, src3, dst3, wf, out, idx_s, idx_d, rows0, rows1, wfv0,
               wfv1, tmp, agg_sh, sem_g0, sem_g1, sem_w0, sem_w1, sem_s0,
               sem_s1):
    c = lax.axis_index("c")
    s = lax.axis_index("s")
    w = c * 16 + s

    def zbody(j, _):
        r = j // 4
        q = j % 4
        tmp[r, pl.ds(q * 16, 16)] = jnp.zeros((16,), jnp.float32)
        return 0
    lax.fori_loop(0, _CPR * 4, zbody, 0)
    row0 = s * _RPT
    for t in range(_RPT // _CPR):
        pltpu.sync_copy(tmp, agg_sh.at[pl.ds(row0 + t * _CPR, _CPR)])
    plsc.subcore_barrier()

    pltpu.sync_copy(src3.at[w], idx_s)
    pltpu.sync_copy(dst3.at[w], idx_d)

    ebase = w * _EPW
    bufs = ((rows0, wfv0, sem_g0, sem_w0, sem_s0),
            (rows1, wfv1, sem_g1, sem_w1, sem_s1))

    def fire(g, b):
        rows_b, wfv_b, sem_gb, sem_wb, _ = bufs[b]
        sbase = ebase + g * _SUP_E
        pltpu.async_copy(wf.at[pl.ds(sbase, _SUP_E), pl.ds(off, _H)], wfv_b,
                         sem_wb)
        for t in range(_SUB):
            pltpu.async_copy(hs.at[idx_s.at[g * _SUB + t]],
                             rows_b.at[pl.ds(t * _CHUNK, _CHUNK)], sem_gb)

    def drain(g, b):
        rows_b, wfv_b, sem_gb, sem_wb, _ = bufs[b]
        sbase = ebase + g * _SUP_E
        for t in range(_SUB):
            pltpu.make_async_copy(hs.at[idx_s.at[g * _SUB + t]],
                                  rows_b.at[pl.ds(t * _CHUNK, _CHUNK)],
                                  sem_gb).wait()
        pltpu.make_async_copy(wf.at[pl.ds(sbase, _SUP_E), pl.ds(off, _H)],
                              wfv_b, sem_wb).wait()

    def mul(b):
        rows_b, wfv_b = bufs[b][0], bufs[b][1]

        def mbody(j, _):
            for r in range(4):
                for q in range(4):
                    sl = pl.ds(q * 16, 16)
                    rows_b[4 * j + r, sl] = (rows_b[4 * j + r, sl]
                                             * wfv_b[4 * j + r, sl])
            return 0
        lax.fori_loop(0, _SUP_E // 4, mbody, 0)

    def fire_scat(g, b):
        rows_b, sem_sb = bufs[b][0], bufs[b][4]
        for t in range(_SUB):
            pltpu.async_copy(rows_b.at[pl.ds(t * _CHUNK, _CHUNK)],
                             agg_sh.at[idx_d.at[g * _SUB + t]], sem_sb,
                             add=True)

    def drain_scat(g, b):
        rows_b, sem_sb = bufs[b][0], bufs[b][4]
        for t in range(_SUB):
            pltpu.make_async_copy(rows_b.at[pl.ds(t * _CHUNK, _CHUNK)],
                                  agg_sh.at[idx_d.at[g * _SUB + t]],
                                  sem_sb).wait()

    fire(0, 0)
    fire(1, 1)

    @pl.loop(0, _NSUP - 2, step=2)
    def _(i2):
        drain(i2, 0)
        mul(0)
        fire_scat(i2, 0)
        drain(i2 + 1, 1)
        mul(1)
        fire_scat(i2 + 1, 1)
        drain_scat(i2, 0)
        fire(i2 + 2, 0)
        drain_scat(i2 + 1, 1)
        fire(i2 + 3, 1)

    drain(_NSUP - 2, 0)
    mul(0)
    fire_scat(_NSUP - 2, 0)
    drain(_NSUP - 1, 1)
    mul(1)
    fire_scat(_NSUP - 1, 1)
    drain_scat(_NSUP - 2, 0)
    drain_scat(_NSUP - 1, 1)
    plsc.subcore_barrier()

    for t in range(_RPT // _CPR):
        pltpu.sync_copy(agg_sh.at[pl.ds(row0 + t * _CPR, _CPR)], tmp)
        pltpu.sync_copy(tmp, out.at[c, pl.ds(row0 + t * _CPR, _CPR)])

  return _sc_body


@functools.cache
def _sc_call(off):
  return pl.kernel(
    _make_sc_body(off),
    out_type=jax.ShapeDtypeStruct((2, _NP, _H), jnp.float32),
    mesh=plsc.VectorSubcoreMesh(core_axis_name="c", subcore_axis_name="s",
                                num_cores=2, num_subcores=16),
    compiler_params=pltpu.CompilerParams(use_tc_tiling_on_sc=False),
    scratch_types=[
        pltpu.VMEM((_NCH, _CHUNK), jnp.int32),
        pltpu.VMEM((_NCH, _CHUNK), jnp.int32),
        pltpu.VMEM((_SUP_E, _H), jnp.float32),
        pltpu.VMEM((_SUP_E, _H), jnp.float32),
        pltpu.VMEM((_SUP_E, _H), jnp.float32),
        pltpu.VMEM((_SUP_E, _H), jnp.float32),
        pltpu.VMEM((_CPR, _H), jnp.float32),
        pltpu.VMEM_SHARED((_NP, _H), jnp.float32),
        pltpu.SemaphoreType.DMA,
        pltpu.SemaphoreType.DMA,
        pltpu.SemaphoreType.DMA,
        pltpu.SemaphoreType.DMA,
        pltpu.SemaphoreType.DMA,
        pltpu.SemaphoreType.DMA,
    ],
  )


def kernel(x, edge_weight, edge_attr, pre_W, pre_b, mlp_W1, mlp_b1, mlp_W2,
           mlp_b2, lin1_W, lin2_W, lin2_b, blk_W, blk_b, bn_g, bn_b, post_W,
           post_b, out_W, out_b, edge_index, batch):
    src3 = edge_index[0].reshape(_NW, _NCH, _CHUNK)
    dst3 = edge_index[1].reshape(_NW, _NCH, _CHUNK)

    w1c = jnp.concatenate([mlp_W1[0], mlp_W1[1], mlp_W1[2]], axis=1)
    b1c = mlp_b1.reshape(1, _L * _H)
    z = jnp.zeros((_H, _H), jnp.float32)
    w2bd = jnp.block([[mlp_W2[0], z, z], [z, mlp_W2[1], z], [z, z, mlp_W2[2]]])
    b2c = mlp_b2.reshape(1, _L * _H)

    wfa, wfb = _edge_call(edge_weight.reshape(1, _E), edge_attr.T, w1c, b1c,
                          w2bd, b2c)
    wf_src = ((wfa, 0), (wfa, _H), (wfb, _H))

    h, hs = _pre_call(x, pre_W, pre_b.reshape(1, _H), lin1_W[0])

    for i in range(_L):
        arr, off = wf_src[i]
        part = _sc_call(off)(hs, src3, dst3, arr)
        args = (h, part, lin2_W[i], lin2_b[i].reshape(1, _H), blk_W[i],
                blk_b[i].reshape(1, _H), bn_g[i].reshape(1, _H),
                bn_b[i].reshape(1, _H))
        if i < _L - 1:
            h, hs = _node_call(*args, lin1_W[i + 1])
        else:
            o = _final_call(*args, batch.reshape(1, _N), post_W,
                            post_b.reshape(1, _H), out_W, out_b.reshape(1, 1))
    return o.reshape(-1)


# speedup vs baseline: 7.1385x; 1.0441x over previous
import functools

import jax
import jax.numpy as jnp
from jax import lax
from jax.experimental import pallas as pl
from jax.experimental.pallas import tpu as pltpu
from jax.experimental.pallas import tpu_sc as plsc

_N = 10000
_E = 320000
_D = 128
_H = 64
_G = 16
_B = 32
_L = 3
_CUTOFF = 8.0
_LOG2 = 0.6931471805599453

_NW = 32
_EPW = _E // _NW
_CHUNK = 40
_SUB = 5
_SUP_E = _CHUNK * _SUB
_NSUP = _EPW // _SUP_E
_NCH = _EPW // _CHUNK
_NP = 10240
_RPT = _NP // 16
_CPR = 64


def _ssp(v):
    return jnp.log(jnp.exp(v) + 1.0) - _LOG2


_BE = 2560


def _make_edge_body(dup):
    def _edge_body(ew_ref, ea_ref, w1_ref, b1_ref, w2_ref, b2_ref, wf_ref):
        c = 0.5 * (jnp.cos(ew_ref[...] * (jnp.pi / _CUTOFF)) + 1.0)
        c = c.reshape(_BE, 1)
        ea = jnp.transpose(ea_ref[...])
        t = (jnp.dot(ea, w1_ref[...], preferred_element_type=jnp.float32)
             + b1_ref[...])
        s = _ssp(t)
        wf = (jnp.dot(s, w2_ref[...], preferred_element_type=jnp.float32)
              + b2_ref[...]) * c
        wf_ref[...] = jnp.concatenate([wf, wf], axis=1) if dup else wf
    return _edge_body


@functools.cache
def _edge_call(width, dup):
    return pl.pallas_call(
        _make_edge_body(dup),
        grid=(_E // _BE,),
        in_specs=[
            pl.BlockSpec((1, _BE), lambda i: (0, i)),
            pl.BlockSpec((_G, _BE), lambda i: (0, i)),
            pl.BlockSpec((_G, width), lambda i: (0, 0)),
            pl.BlockSpec((1, width), lambda i: (0, 0)),
            pl.BlockSpec((width, width), lambda i: (0, 0)),
            pl.BlockSpec((1, width), lambda i: (0, 0)),
        ],
        out_specs=pl.BlockSpec((_BE, 2 * _H), lambda i: (i, 0)),
        out_shape=jax.ShapeDtypeStruct((_E, 2 * _H), jnp.float32),
    )


def _pre_body(x_ref, pw_ref, pb_ref, l1_ref, h_ref, hs_ref):
    h = jax.nn.relu(jnp.dot(x_ref[...], pw_ref[...],
                            preferred_element_type=jnp.float32) + pb_ref[...])
    h_ref[...] = h
    hs_ref[...] = jnp.dot(h, l1_ref[...], preferred_element_type=jnp.float32)


_pre_call = pl.pallas_call(
    _pre_body,
    out_shape=[jax.ShapeDtypeStruct((_N, _H), jnp.float32)] * 2,
)


def _node_update(h, part, l2w, l2b, bw, bb, g, b):
    agg = (part[0] + part[1])[:_N]
    c = _ssp(jnp.dot(agg, l2w, preferred_element_type=jnp.float32) + l2b)
    c = jnp.dot(c, bw, preferred_element_type=jnp.float32) + bb
    hn = h + c
    mu = jnp.mean(hn, axis=0, keepdims=True)
    var = jnp.mean((hn - mu) ** 2, axis=0, keepdims=True)
    return (hn - mu) * jax.lax.rsqrt(var + 1e-5) * g + b


def _node_body(h_ref, part_ref, l2w_ref, l2b_ref, bw_ref, bb_ref,
               g_ref, b_ref, l1n_ref, h_out, hs_out):
    hn = _node_update(h_ref[...], part_ref[...], l2w_ref[...], l2b_ref[...],
                     bw_ref[...], bb_ref[...], g_ref[...], b_ref[...])
    h_out[...] = hn
    hs_out[...] = jnp.dot(hn, l1n_ref[...], preferred_element_type=jnp.float32)


_node_call = pl.pallas_call(
    _node_body,
    out_shape=[jax.ShapeDtypeStruct((_N, _H), jnp.float32)] * 2,
)


def _final_body(h_ref, part_ref, l2w_ref, l2b_ref, bw_ref, bb_ref,
                g_ref, b_ref, batch_ref, pw_ref, pb_ref, ow_ref, ob_ref,
                o_ref):
    hn = _node_update(h_ref[...], part_ref[...], l2w_ref[...], l2b_ref[...],
                     bw_ref[...], bb_ref[...], g_ref[...], b_ref[...])
    seg = lax.broadcasted_iota(jnp.int32, (_B, _N), 0)
    oht = (seg == batch_ref[...]).astype(jnp.float32)
    counts = jnp.sum(oht, axis=1, keepdims=True)
    pooled = jnp.dot(oht, hn, preferred_element_type=jnp.float32)
    pooled = pooled / jnp.maximum(counts, 1.0)
    o = jax.nn.relu(jnp.dot(pooled, pw_ref[...],
                            preferred_element_type=jnp.float32) + pb_ref[...])
    o_ref[...] = jnp.dot(o, ow_ref[...], preferred_element_type=jnp.float32) + ob_ref[...]


_final_call = pl.pallas_call(
    _final_body,
    out_shape=jax.ShapeDtypeStruct((_B, 1), jnp.float32),
)


def _make_sc_body(off):
  def _sc_body(hs, src3, dst3, wf, out, idx_s, idx_d, rows0, rows1, wfv0,
               wfv1, tmp, agg_sh, sem_g0, sem_g1, sem_w0, sem_w1, sem_s0,
               sem_s1):
    c = lax.axis_index("c")
    s = lax.axis_index("s")
    w = c * 16 + s

    def zbody(j, _):
        r = j // 4
        q = j % 4
        tmp[r, pl.ds(q * 16, 16)] = jnp.zeros((16,), jnp.float32)
        return 0
    lax.fori_loop(0, _CPR * 4, zbody, 0)
    row0 = s * _RPT
    for t in range(_RPT // _CPR):
        pltpu.sync_copy(tmp, agg_sh.at[pl.ds(row0 + t * _CPR, _CPR)])
    plsc.subcore_barrier()

    pltpu.sync_copy(src3.at[w], idx_s)
    pltpu.sync_copy(dst3.at[w], idx_d)

    ebase = w * _EPW
    bufs = ((rows0, wfv0, sem_g0, sem_w0, sem_s0),
            (rows1, wfv1, sem_g1, sem_w1, sem_s1))

    def fire(g, b):
        rows_b, wfv_b, sem_gb, sem_wb, _ = bufs[b]
        sbase = ebase + g * _SUP_E
        pltpu.async_copy(wf.at[pl.ds(sbase, _SUP_E), pl.ds(off, _H)], wfv_b,
                         sem_wb)
        for t in range(_SUB):
            pltpu.async_copy(hs.at[idx_s.at[g * _SUB + t]],
                             rows_b.at[pl.ds(t * _CHUNK, _CHUNK)], sem_gb)

    def drain(g, b):
        rows_b, wfv_b, sem_gb, sem_wb, _ = bufs[b]
        sbase = ebase + g * _SUP_E
        for t in range(_SUB):
            pltpu.make_async_copy(hs.at[idx_s.at[g * _SUB + t]],
                                  rows_b.at[pl.ds(t * _CHUNK, _CHUNK)],
                                  sem_gb).wait()
        pltpu.make_async_copy(wf.at[pl.ds(sbase, _SUP_E), pl.ds(off, _H)],
                              wfv_b, sem_wb).wait()

    def mul(b):
        rows_b, wfv_b = bufs[b][0], bufs[b][1]

        def mbody(j, _):
            for r in range(4):
                for q in range(4):
                    sl = pl.ds(q * 16, 16)
                    rows_b[4 * j + r, sl] = (rows_b[4 * j + r, sl]
                                             * wfv_b[4 * j + r, sl])
            return 0
        lax.fori_loop(0, _SUP_E // 4, mbody, 0)

    def fire_scat(g, b):
        rows_b, sem_sb = bufs[b][0], bufs[b][4]
        for t in range(_SUB):
            pltpu.async_copy(rows_b.at[pl.ds(t * _CHUNK, _CHUNK)],
                             agg_sh.at[idx_d.at[g * _SUB + t]], sem_sb,
                             add=True)

    def drain_scat(g, b):
        rows_b, sem_sb = bufs[b][0], bufs[b][4]
        for t in range(_SUB):
            pltpu.make_async_copy(rows_b.at[pl.ds(t * _CHUNK, _CHUNK)],
                                  agg_sh.at[idx_d.at[g * _SUB + t]],
                                  sem_sb).wait()

    fire(0, 0)
    fire(1, 1)

    @pl.loop(0, _NSUP - 2, step=2)
    def _(i2):
        drain(i2, 0)
        mul(0)
        fire_scat(i2, 0)
        drain(i2 + 1, 1)
        mul(1)
        fire_scat(i2 + 1, 1)
        drain_scat(i2, 0)
        fire(i2 + 2, 0)
        drain_scat(i2 + 1, 1)
        fire(i2 + 3, 1)

    drain(_NSUP - 2, 0)
    mul(0)
    fire_scat(_NSUP - 2, 0)
    drain(_NSUP - 1, 1)
    mul(1)
    fire_scat(_NSUP - 1, 1)
    drain_scat(_NSUP - 2, 0)
    drain_scat(_NSUP - 1, 1)
    plsc.subcore_barrier()

    for t in range(_RPT // _CPR):
        pltpu.sync_copy(agg_sh.at[pl.ds(row0 + t * _CPR, _CPR)], tmp)
        pltpu.sync_copy(tmp, out.at[c, pl.ds(row0 + t * _CPR, _CPR)])

  return _sc_body


@functools.cache
def _sc_call(off):
  return pl.kernel(
    _make_sc_body(off),
    out_type=jax.ShapeDtypeStruct((2, _NP, _H), jnp.float32),
    mesh=plsc.VectorSubcoreMesh(core_axis_name="c", subcore_axis_name="s",
                                num_cores=2, num_subcores=16),
    compiler_params=pltpu.CompilerParams(use_tc_tiling_on_sc=False),
    scratch_types=[
        pltpu.VMEM((_NCH, _CHUNK), jnp.int32),
        pltpu.VMEM((_NCH, _CHUNK), jnp.int32),
        pltpu.VMEM((_SUP_E, _H), jnp.float32),
        pltpu.VMEM((_SUP_E, _H), jnp.float32),
        pltpu.VMEM((_SUP_E, _H), jnp.float32),
        pltpu.VMEM((_SUP_E, _H), jnp.float32),
        pltpu.VMEM((_CPR, _H), jnp.float32),
        pltpu.VMEM_SHARED((_NP, _H), jnp.float32),
        pltpu.SemaphoreType.DMA,
        pltpu.SemaphoreType.DMA,
        pltpu.SemaphoreType.DMA,
        pltpu.SemaphoreType.DMA,
        pltpu.SemaphoreType.DMA,
        pltpu.SemaphoreType.DMA,
    ],
  )


def kernel(x, edge_weight, edge_attr, pre_W, pre_b, mlp_W1, mlp_b1, mlp_W2,
           mlp_b2, lin1_W, lin2_W, lin2_b, blk_W, blk_b, bn_g, bn_b, post_W,
           post_b, out_W, out_b, edge_index, batch):
    src3 = edge_index[0].reshape(_NW, _NCH, _CHUNK)
    dst3 = edge_index[1].reshape(_NW, _NCH, _CHUNK)

    ew1r = edge_weight.reshape(1, _E)
    eat = edge_attr.T
    w1ab = jnp.concatenate([mlp_W1[0], mlp_W1[1]], axis=1)
    b1ab = mlp_b1[0:2].reshape(1, 2 * _H)
    z = jnp.zeros((_H, _H), jnp.float32)
    w2ab = jnp.block([[mlp_W2[0], z], [z, mlp_W2[1]]])
    b2ab = mlp_b2[0:2].reshape(1, 2 * _H)
    wf_a = _edge_call(2 * _H, False)(ew1r, eat, w1ab, b1ab, w2ab, b2ab)
    wf_b = _edge_call(_H, True)(ew1r, eat, mlp_W1[2],
                                mlp_b1[2].reshape(1, _H), mlp_W2[2],
                                mlp_b2[2].reshape(1, _H))
    wf_src = ((wf_a, 0), (wf_a, _H), (wf_b, 0))

    h, hs = _pre_call(x, pre_W, pre_b.reshape(1, _H), lin1_W[0])

    for i in range(_L):
        arr, off = wf_src[i]
        part = _sc_call(off)(hs, src3, dst3, arr)
        args = (h, part, lin2_W[i], lin2_b[i].reshape(1, _H), blk_W[i],
                blk_b[i].reshape(1, _H), bn_g[i].reshape(1, _H),
                bn_b[i].reshape(1, _H))
        if i < _L - 1:
            h, hs = _node_call(*args, lin1_W[i + 1])
        else:
            o = _final_call(*args, batch.reshape(1, _N), post_W,
                            post_b.reshape(1, _H), out_W, out_b.reshape(1, 1))
    return o.reshape(-1)
